# baseline (speedup 1.0000x reference)
import functools

import numpy as np

import jax
import jax.numpy as jnp
from jax import lax
from jax.experimental import pallas as pl
from jax.experimental.pallas import tpu as pltpu
from jax.experimental.pallas import tpu_sc as plsc

NN = 10000
NE = 320000
ND = 128
NW = 32
EPW = NE // NW
CK = 40
NCK = EPW // CK
NNP = 10240
RPT = NNP // 16
ZB = CK

_CPRM = np.concatenate([
    32 * j + np.stack([np.arange(16), 16 + np.arange(16)], axis=1).reshape(-1)
    for j in range(ND // 32)
])


def _mm_nodes_body(x_ref, wa_ref, wb_ref, a_ref, b_ref):
    xb = x_ref[...]
    a_ref[...] = jnp.dot(xb, wa_ref[...], preferred_element_type=jnp.float32)
    b_ref[...] = jnp.dot(xb, wb_ref[...], preferred_element_type=jnp.float32)


_mm_nodes = pl.pallas_call(
    _mm_nodes_body,
    grid=(10,),
    in_specs=[
        pl.BlockSpec((NN // 10, ND), lambda i: (i, 0)),
        pl.BlockSpec((ND, ND), lambda i: (0, 0)),
        pl.BlockSpec((ND, ND), lambda i: (0, 0)),
    ],
    out_specs=[
        pl.BlockSpec((NN // 10, ND), lambda i: (i, 0)),
        pl.BlockSpec((NN // 10, ND), lambda i: (i, 0)),
    ],
    out_shape=[jax.ShapeDtypeStruct((NN, ND), jnp.float32)] * 2,
)


def _mm_edges_body(ea_ref, rq_ref, w_ref, b_ref, c_ref, dg_ref):
    c_ref[...] = (
        jnp.dot(ea_ref[...], w_ref[...], preferred_element_type=jnp.float32)
        + b_ref[0:1, :]
    ).astype(jnp.bfloat16)
    r = rq_ref[...]
    q = r // ND
    m = r % ND
    ohq = (q == lax.broadcasted_iota(jnp.int32, (_EBLK, NNP // ND), 1)).astype(
        jnp.float32
    )
    ohm = (m == lax.broadcasted_iota(jnp.int32, (_EBLK, ND), 1)).astype(
        jnp.float32
    )
    part = lax.dot_general(
        ohq, ohm, (((0,), (0,)), ((), ())), preferred_element_type=jnp.float32
    )

    @pl.when(pl.program_id(0) == 0)
    def _init():
        dg_ref[...] = jnp.zeros_like(dg_ref)

    dg_ref[...] += part


_EBLK = 4000
_mm_edges = pl.pallas_call(
    _mm_edges_body,
    grid=(NE // _EBLK,),
    in_specs=[
        pl.BlockSpec((_EBLK, 16), lambda i: (i, 0)),
        pl.BlockSpec((_EBLK, 1), lambda i: (i, 0)),
        pl.BlockSpec((16, ND), lambda i: (0, 0)),
        pl.BlockSpec((8, ND), lambda i: (0, 0)),
    ],
    out_specs=[
        pl.BlockSpec((_EBLK, ND), lambda i: (i, 0)),
        pl.BlockSpec((NNP // ND, ND), lambda i: (0, 0)),
    ],
    out_shape=[
        jax.ShapeDtypeStruct((NE, ND), jnp.bfloat16),
        jax.ShapeDtypeStruct((NNP // ND, ND), jnp.float32),
    ],
)


def _mm_out_body(hp_ref, deg_ref, w_ref, b2_ref, o_ref):
    h = hp_ref[0] + hp_ref[1]
    d = deg_ref[0]
    degb = lax.dot_general(
        d, b2_ref[0:1, :], (((0,), (0,)), ((), ())),
        preferred_element_type=jnp.float32,
    )
    o_ref[...] = (
        jnp.dot(h, w_ref[...], preferred_element_type=jnp.float32) + degb
    )


_mm_out = pl.pallas_call(
    _mm_out_body,
    grid=(NNP // ND,),
    in_specs=[
        pl.BlockSpec((2, ND, ND), lambda i: (0, i, 0)),
        pl.BlockSpec((1, 1, ND), lambda i: (i, 0, 0)),
        pl.BlockSpec((ND, ND), lambda i: (0, 0)),
        pl.BlockSpec((8, ND), lambda i: (0, 0)),
    ],
    out_specs=pl.BlockSpec((ND, ND), lambda i: (i, 0)),
    out_shape=jax.ShapeDtypeStruct((NNP, ND), jnp.float32),
)


@functools.cache
def _build_sc_edges():
    mesh = plsc.VectorSubcoreMesh(core_axis_name="c", subcore_axis_name="s")
    return functools.partial(
        pl.kernel,
        mesh=mesh,
        out_type=jax.ShapeDtypeStruct((2, NNP, ND), jnp.float32),
        scratch_types=[
        pltpu.VMEM((2, 2, CK), jnp.int32),
        pltpu.VMEM((2, CK), jnp.int32),
        pltpu.VMEM((2, CK, ND), jnp.float32),
        pltpu.VMEM((2, CK, ND), jnp.float32),
        pltpu.VMEM((2, CK, ND // 2), jnp.int32),
        pltpu.VMEM_SHARED((NNP, ND), jnp.float32),
        pltpu.SemaphoreType.DMA,
        pltpu.SemaphoreType.DMA,
        pltpu.SemaphoreType.DMA,
        pltpu.SemaphoreType.DMA,
        pltpu.SemaphoreType.DMA,
        pltpu.SemaphoreType.DMA,
        pltpu.SemaphoreType.DMA,
        pltpu.SemaphoreType.DMA,
        ],
    )(_sc_edges_body)


def _sc_edges_body(a_hbm, b_hbm, c_hbm, idx_hbm, row_hbm, out_hbm,
                   idxv, sidx, av2, bv2, cv2, acc,
                   sem_g0, sem_g1, sem_s0, sem_s1, sem_i0, sem_i1,
                   sem_t0, sem_t1):
    cid = lax.axis_index("c")
    sid = lax.axis_index("s")
    wid = sid * 2 + cid
    base = wid * EPW
    sems_g = (sem_g0, sem_g1)
    sems_s = (sem_s0, sem_s1)
    sems_i = (sem_i0, sem_i1)
    sems_t = (sem_t0, sem_t1)
    zb = bv2.at[0]

    def _zero(i, carry):
        for j in range(ND // 16):
            zb[i, pl.ds(j * 16, 16)] = jnp.zeros((16,), jnp.float32)
        return carry

    lax.fori_loop(0, ZB, _zero, 0)
    for r in range(RPT // ZB):
        pltpu.sync_copy(zb, acc.at[pl.ds(sid * RPT + r * ZB, ZB)])
    plsc.subcore_barrier()

    def fetch_idx(c, b):
        pltpu.async_copy(idx_hbm.at[wid, c], idxv.at[b], sems_i[b])

    def wait_i(b):
        pltpu.make_async_copy(idx_hbm.at[wid, 0], idxv.at[b], sems_i[b]).wait()

    def fetch_sidx(c, b):
        pltpu.async_copy(row_hbm.at[wid, c], sidx.at[b], sems_t[b])

    def wait_t(b):
        pltpu.make_async_copy(row_hbm.at[wid, 0], sidx.at[b], sems_t[b]).wait()

    def launch(c, b):
        pltpu.async_copy(a_hbm.at[idxv.at[b, 0]], av2.at[b], sems_g[b])
        pltpu.async_copy(b_hbm.at[idxv.at[b, 1]], bv2.at[b], sems_g[b])
        pltpu.async_copy(c_hbm.at[pl.ds(base + c * CK, CK)], cv2.at[b], sems_g[b])

    def wait_g(b):
        pltpu.make_async_copy(a_hbm.at[idxv.at[b, 0]], av2.at[b], sems_g[b]).wait()
        pltpu.make_async_copy(b_hbm.at[idxv.at[b, 1]], bv2.at[b], sems_g[b]).wait()
        pltpu.make_async_copy(c_hbm.at[pl.ds(base, CK)], cv2.at[b], sems_g[b]).wait()

    def wait_s(b):
        pltpu.make_async_copy(av2.at[b], acc.at[sidx.at[b]], sems_s[b]).wait()

    def compute(b):
        avb, bvb, cvb = av2.at[b], bv2.at[b], cv2.at[b]

        himask = jnp.full((16,), -65536, jnp.int32)

        @plsc.parallel_loop(0, CK, unroll=4)
        def _edge(e):
            for j in range(ND // 32):
                ci = cvb[e, pl.ds(j * 16, 16)]
                c0 = lax.bitcast_convert_type(
                    jnp.left_shift(ci, 16), jnp.float32)
                c1 = lax.bitcast_convert_type(
                    jnp.bitwise_and(ci, himask), jnp.float32)
                s0 = pl.ds(j * 32, 16)
                s1 = pl.ds(j * 32 + 16, 16)
                avb[e, s0] = jnp.maximum(avb[e, s0] + bvb[e, s0] + c0, 0.0)
                avb[e, s1] = jnp.maximum(avb[e, s1] + bvb[e, s1] + c1, 0.0)

    def scatter(b):
        pltpu.async_copy(av2.at[b], acc.at[sidx.at[b]], sems_s[b], add=True)

    def slot(c, b, nb, do_fetch, do_wait_s, do_launch):
        wait_g(b)
        fetch_sidx(c, b)
        if do_fetch:
            fetch_idx(c + 2, b)
        if do_wait_s:
            wait_s(nb)
        if do_launch:
            wait_i(nb)
            launch(c + 1, nb)
        compute(b)
        wait_t(b)
        scatter(b)

    pltpu.sync_copy(idx_hbm.at[wid, 0], idxv.at[0])
    fetch_idx(1, 1)
    launch(0, 0)
    slot(0, 0, 1, True, False, True)

    def _pair(k, carry):
        c1 = 2 * k + 1
        slot(c1, 1, 0, True, True, True)
        slot(c1 + 1, 0, 1, True, True, True)
        return carry

    lax.fori_loop(0, (NCK - 2) // 2, _pair, 0)

    slot(NCK - 1, 1, 0, False, True, False)
    wait_i(0)
    wait_s(1)
    plsc.subcore_barrier()

    for r in range(RPT // ZB):
        rows = pl.ds(sid * RPT + r * ZB, ZB)
        pltpu.sync_copy(acc.at[rows], zb)
        pltpu.sync_copy(zb, out_hbm.at[cid].at[rows])


def kernel(x, edge_index, edge_attr, W1, b1, W2, b2):
    row = edge_index[0].astype(jnp.int32)
    col = edge_index[1].astype(jnp.int32)
    w1a = W1[:ND]
    w1b = W1[ND:2 * ND]
    w1c = W1[2 * ND:][:, _CPRM]
    b1x8 = jnp.broadcast_to(b1[_CPRM], (8, ND))
    b2x8 = jnp.broadcast_to(b2, (8, ND))
    a_tab, b_tab = _mm_nodes(x, w1a, w1b)
    c_tab, degm = _mm_edges(edge_attr, row[:, None], w1c, b1x8)
    row3 = row.reshape(NW, NCK, CK)
    col3 = col.reshape(NW, NCK, CK)
    idx3 = jnp.stack([row3, col3], axis=2)
    idx3 = jnp.pad(idx3, ((0, 0), (0, 1), (0, 0), (0, 0)))
    c_tab = lax.bitcast_convert_type(c_tab.reshape(NE, ND // 2, 2), jnp.int32)
    hp = _build_sc_edges()(a_tab, b_tab, c_tab, idx3, row3)
    deg3 = degm.reshape(NNP // ND, 1, ND)
    return _mm_out(hp, deg3, W2, b2x8)[:NN]


# speedup vs baseline: 1.9715x; 1.9715x over previous
import functools

import jax
import jax.numpy as jnp
from jax import lax
from jax.experimental import pallas as pl
from jax.experimental.pallas import tpu as pltpu
from jax.experimental.pallas import tpu_sc as plsc

NN = 10000
NE = 320000
ND = 128
NW = 32
EPW = NE // NW
CK = 40
NCK = EPW // CK
NNP = 10240
RPT = NNP // 16
ZB = CK


def _mm_nodes_body(x_ref, wa_ref, wb_ref, a_ref, b_ref):
    xb = x_ref[...]
    a_ref[...] = jnp.dot(xb, wa_ref[...], preferred_element_type=jnp.float32)
    b_ref[...] = jnp.dot(xb, wb_ref[...], preferred_element_type=jnp.float32)


_mm_nodes = pl.pallas_call(
    _mm_nodes_body,
    grid=(10,),
    in_specs=[
        pl.BlockSpec((NN // 10, ND), lambda i: (i, 0)),
        pl.BlockSpec((ND, ND), lambda i: (0, 0)),
        pl.BlockSpec((ND, ND), lambda i: (0, 0)),
    ],
    out_specs=[
        pl.BlockSpec((NN // 10, ND), lambda i: (i, 0)),
        pl.BlockSpec((NN // 10, ND), lambda i: (i, 0)),
    ],
    out_shape=[jax.ShapeDtypeStruct((NN, ND), jnp.float32)] * 2,
)


def _mm_edges_body(ea_ref, rq_ref, w_ref, b_ref, c_ref, dg_ref):
    c_ref[...] = (
        jnp.dot(ea_ref[...], w_ref[...], preferred_element_type=jnp.float32)
        + b_ref[0:1, :]
    )
    r = rq_ref[...]
    q = r // ND
    m = r % ND
    ohq = (q == lax.broadcasted_iota(jnp.int32, (_EBLK, NNP // ND), 1)).astype(
        jnp.float32
    )
    ohm = (m == lax.broadcasted_iota(jnp.int32, (_EBLK, ND), 1)).astype(
        jnp.float32
    )
    part = lax.dot_general(
        ohq, ohm, (((0,), (0,)), ((), ())), preferred_element_type=jnp.float32
    )

    @pl.when(pl.program_id(0) == 0)
    def _init():
        dg_ref[...] = jnp.zeros_like(dg_ref)

    dg_ref[...] += part


_EBLK = 4000
_mm_edges = pl.pallas_call(
    _mm_edges_body,
    grid=(NE // _EBLK,),
    in_specs=[
        pl.BlockSpec((_EBLK, 16), lambda i: (i, 0)),
        pl.BlockSpec((_EBLK, 1), lambda i: (i, 0)),
        pl.BlockSpec((16, ND), lambda i: (0, 0)),
        pl.BlockSpec((8, ND), lambda i: (0, 0)),
    ],
    out_specs=[
        pl.BlockSpec((_EBLK, ND), lambda i: (i, 0)),
        pl.BlockSpec((NNP // ND, ND), lambda i: (0, 0)),
    ],
    out_shape=[
        jax.ShapeDtypeStruct((NE, ND), jnp.float32),
        jax.ShapeDtypeStruct((NNP // ND, ND), jnp.float32),
    ],
)


def _mm_out_body(hp_ref, deg_ref, w_ref, b2_ref, o_ref):
    h = hp_ref[0] + hp_ref[1]
    d = deg_ref[0]
    degb = lax.dot_general(
        d, b2_ref[0:1, :], (((0,), (0,)), ((), ())),
        preferred_element_type=jnp.float32,
    )
    o_ref[...] = (
        jnp.dot(h, w_ref[...], preferred_element_type=jnp.float32) + degb
    )


_mm_out = pl.pallas_call(
    _mm_out_body,
    grid=(NNP // ND,),
    in_specs=[
        pl.BlockSpec((2, ND, ND), lambda i: (0, i, 0)),
        pl.BlockSpec((1, 1, ND), lambda i: (i, 0, 0)),
        pl.BlockSpec((ND, ND), lambda i: (0, 0)),
        pl.BlockSpec((8, ND), lambda i: (0, 0)),
    ],
    out_specs=pl.BlockSpec((ND, ND), lambda i: (i, 0)),
    out_shape=jax.ShapeDtypeStruct((NNP, ND), jnp.float32),
)


@functools.cache
def _build_sc_edges():
    mesh = plsc.VectorSubcoreMesh(core_axis_name="c", subcore_axis_name="s")
    return functools.partial(
        pl.kernel,
        mesh=mesh,
        out_type=jax.ShapeDtypeStruct((2, NNP, ND), jnp.float32),
        scratch_types=[
        pltpu.VMEM((2, 2, CK), jnp.int32),
        pltpu.VMEM((2, CK), jnp.int32),
        pltpu.VMEM((2, CK, ND), jnp.float32),
        pltpu.VMEM((2, CK, ND), jnp.float32),
        pltpu.VMEM((2, CK, ND), jnp.float32),
        pltpu.VMEM_SHARED((NNP, ND), jnp.float32),
        pltpu.SemaphoreType.DMA,
        pltpu.SemaphoreType.DMA,
        pltpu.SemaphoreType.DMA,
        pltpu.SemaphoreType.DMA,
        pltpu.SemaphoreType.DMA,
        pltpu.SemaphoreType.DMA,
        pltpu.SemaphoreType.DMA,
        pltpu.SemaphoreType.DMA,
        ],
    )(_sc_edges_body)


def _sc_edges_body(a_hbm, b_hbm, c_hbm, idx_hbm, row_hbm, out_hbm,
                   idxv, sidx, av2, bv2, cv2, acc,
                   sem_g0, sem_g1, sem_s0, sem_s1, sem_i0, sem_i1,
                   sem_t0, sem_t1):
    cid = lax.axis_index("c")
    sid = lax.axis_index("s")
    wid = sid * 2 + cid
    base = wid * EPW
    sems_g = (sem_g0, sem_g1)
    sems_s = (sem_s0, sem_s1)
    sems_i = (sem_i0, sem_i1)
    sems_t = (sem_t0, sem_t1)
    zb = bv2.at[0]

    def _zero(i, carry):
        for j in range(ND // 16):
            zb[i, pl.ds(j * 16, 16)] = jnp.zeros((16,), jnp.float32)
        return carry

    lax.fori_loop(0, ZB, _zero, 0)
    for r in range(RPT // ZB):
        pltpu.sync_copy(zb, acc.at[pl.ds(sid * RPT + r * ZB, ZB)])
    plsc.subcore_barrier()

    def fetch_idx(c, b):
        pltpu.async_copy(idx_hbm.at[wid, c], idxv.at[b], sems_i[b])

    def wait_i(b):
        pltpu.make_async_copy(idx_hbm.at[wid, 0], idxv.at[b], sems_i[b]).wait()

    def fetch_sidx(c, b):
        pltpu.async_copy(row_hbm.at[wid, c], sidx.at[b], sems_t[b])

    def wait_t(b):
        pltpu.make_async_copy(row_hbm.at[wid, 0], sidx.at[b], sems_t[b]).wait()

    def launch(c, b):
        pltpu.async_copy(a_hbm.at[idxv.at[b, 0]], av2.at[b], sems_g[b])
        pltpu.async_copy(b_hbm.at[idxv.at[b, 1]], bv2.at[b], sems_g[b])
        pltpu.async_copy(c_hbm.at[pl.ds(base + c * CK, CK)], cv2.at[b], sems_g[b])

    def wait_g(b):
        pltpu.make_async_copy(a_hbm.at[idxv.at[b, 0]], av2.at[b], sems_g[b]).wait()
        pltpu.make_async_copy(b_hbm.at[idxv.at[b, 1]], bv2.at[b], sems_g[b]).wait()
        pltpu.make_async_copy(c_hbm.at[pl.ds(base, CK)], cv2.at[b], sems_g[b]).wait()

    def wait_s(b):
        pltpu.make_async_copy(av2.at[b], acc.at[sidx.at[b]], sems_s[b]).wait()

    def compute(b):
        avb, bvb, cvb = av2.at[b], bv2.at[b], cv2.at[b]

        @plsc.parallel_loop(0, CK, unroll=4)
        def _edge(e):
            for j in range(ND // 16):
                s = pl.ds(j * 16, 16)
                avb[e, s] = jnp.maximum(avb[e, s] + bvb[e, s] + cvb[e, s], 0.0)

    def scatter(b):
        pltpu.async_copy(av2.at[b], acc.at[sidx.at[b]], sems_s[b], add=True)

    def slot(c, b, nb, do_fetch, do_wait_s, do_launch):
        wait_g(b)
        fetch_sidx(c, b)
        if do_fetch:
            fetch_idx(c + 2, b)
        if do_wait_s:
            wait_s(nb)
        if do_launch:
            wait_i(nb)
            launch(c + 1, nb)
        compute(b)
        wait_t(b)
        scatter(b)

    pltpu.sync_copy(idx_hbm.at[wid, 0], idxv.at[0])
    fetch_idx(1, 1)
    launch(0, 0)
    slot(0, 0, 1, True, False, True)

    def _pair(k, carry):
        c1 = 2 * k + 1
        slot(c1, 1, 0, True, True, True)
        slot(c1 + 1, 0, 1, True, True, True)
        return carry

    lax.fori_loop(0, (NCK - 2) // 2, _pair, 0)

    slot(NCK - 1, 1, 0, False, True, False)
    wait_i(0)
    wait_s(1)
    plsc.subcore_barrier()

    for r in range(RPT // ZB):
        rows = pl.ds(sid * RPT + r * ZB, ZB)
        pltpu.sync_copy(acc.at[rows], zb)
        pltpu.sync_copy(zb, out_hbm.at[cid].at[rows])


def kernel(x, edge_index, edge_attr, W1, b1, W2, b2):
    row = edge_index[0].astype(jnp.int32)
    col = edge_index[1].astype(jnp.int32)
    w1a = W1[:ND]
    w1b = W1[ND:2 * ND]
    w1c = W1[2 * ND:]
    b1x8 = jnp.broadcast_to(b1, (8, ND))
    b2x8 = jnp.broadcast_to(b2, (8, ND))
    a_tab, b_tab = _mm_nodes(x, w1a, w1b)
    c_tab, degm = _mm_edges(edge_attr, row[:, None], w1c, b1x8)
    row3 = row.reshape(NW, NCK, CK)
    col3 = col.reshape(NW, NCK, CK)
    idx3 = jnp.stack([row3, col3], axis=2)
    idx3 = jnp.pad(idx3, ((0, 0), (0, 1), (0, 0), (0, 0)))
    hp = _build_sc_edges()(a_tab, b_tab, c_tab, idx3, row3)
    deg3 = degm.reshape(NNP // ND, 1, ND)
    return _mm_out(hp, deg3, W2, b2x8)[:NN]
